# baseline (speedup 1.0000x reference)
import functools

import jax
import jax.numpy as jnp
import numpy as np
from jax import lax
from jax.experimental import pallas as pl
from jax.experimental.pallas import tpu as pltpu
from jax.experimental.pallas import tpu_sc as plsc

N = 10000
E = 320000
H = 128
L = 4
G = 500
GP = 512
ZT = 1024
R = 400
NB = N // R

NT = 16
NC = 2
C = 64
EP_PAD = 327680
HALF = EP_PAD // 2
EPT0 = 18432
EPT1 = 2048
K = 16
NPD = 10240
RPT = NPD // NT
ZR = 64
BNS = float(1.0 / np.sqrt(1.0 + 1e-5))
EPS = 1e-5


def _edge_body(a, ep, src2, dst2, out0, out1,
               agg_s, sidx, didx, gb0, gb1, pb0, pb1,
               sem_i, sem_g0, sem_g1, sem_e0, sem_e1, sem_s0, sem_s1):
    c = lax.axis_index("c")
    s = lax.axis_index("s")
    gbufs = (gb0, gb1)
    pbufs = (pb0, pb1)
    gsems = (sem_g0, sem_g1)
    esems = (sem_e0, sem_e1)
    ssems = (sem_s0, sem_s1)

    @pl.loop(0, ZR)
    def _zero(i):
        for k in range(8):
            gb0[i, pl.ds(k * 16, 16)] = jnp.zeros((16,), jnp.float32)

    for i in range(RPT // ZR):
        pltpu.sync_copy(gb0.at[pl.ds(0, ZR)],
                        agg_s.at[pl.ds(s * RPT + i * ZR, ZR)])
    plsc.subcore_barrier()

    ebase0 = s * EPT0
    ebase1 = NT * EPT0 + s * EPT1
    ebase = pl.multiple_of(jnp.where(c == 0, ebase0, ebase1), 512)
    nsup = jnp.where(c == 0, EPT0 // (K * C), EPT1 // (K * C))

    def issue_loads(sup, i, b):
        pltpu.async_copy(a.at[sidx.at[i]], gbufs[b], gsems[b])
        pltpu.async_copy(ep.at[pl.ds(ebase + (sup * K + i) * C, C)],
                         pbufs[b], esems[b])

    def wait_loads(b):
        pltpu.make_async_copy(a.at[sidx.at[0]], gbufs[b], gsems[b]).wait()
        pltpu.make_async_copy(ep.at[pl.ds(0, C)], pbufs[b],
                              esems[b]).wait()

    def relu(b):
        gbuf, pbuf = gbufs[b], pbufs[b]

        @pl.loop(0, C)
        def _relu(j):
            for k in range(8):
                sl = pl.ds(k * 16, 16)
                pbuf[j, sl] = jnp.maximum(gbuf[j, sl] + pbuf[j, sl], 0.0)

    def issue_scatter(i, b):
        pltpu.async_copy(pbufs[b], agg_s.at[didx.at[i]], ssems[b], add=True)

    def wait_scatter(b):
        pltpu.make_async_copy(pbufs[b], agg_s.at[didx.at[0]], ssems[b]).wait()

    @pl.loop(0, nsup)
    def _sup(sup):
        irow = pl.multiple_of(ebase // C + sup * K, 8)
        pltpu.sync_copy(src2.at[pl.ds(irow, K)], sidx)
        pltpu.sync_copy(dst2.at[pl.ds(irow, K)], didx)

        issue_loads(sup, 0, 0)
        issue_loads(sup, 1, 1)

        @pl.loop(0, K // 2 - 1)
        def _chunk(ii):
            i = ii * 2
            wait_loads(0)
            relu(0)
            issue_scatter(i, 0)
            wait_loads(1)
            relu(1)
            issue_scatter(i + 1, 1)
            wait_scatter(0)
            issue_loads(sup, i + 2, 0)
            wait_scatter(1)
            issue_loads(sup, i + 3, 1)

        wait_loads(0)
        relu(0)
        issue_scatter(K - 2, 0)
        wait_loads(1)
        relu(1)
        issue_scatter(K - 1, 1)
        wait_scatter(0)
        wait_scatter(1)

    plsc.subcore_barrier()

    @pl.when(c == 0)
    def _o0():
        pltpu.sync_copy(agg_s.at[pl.ds(s * RPT, RPT)],
                        out0.at[pl.ds(s * RPT, RPT)])

    @pl.when(c == 1)
    def _o1():
        pltpu.sync_copy(agg_s.at[pl.ds(s * RPT, RPT)],
                        out1.at[pl.ds(s * RPT, RPT)])


@jax.jit
def _edge_sc(a, ep, src2, dst2):
    mesh = plsc.VectorSubcoreMesh(core_axis_name="c", subcore_axis_name="s")
    fn = pl.kernel(
        _edge_body,
        out_type=(jax.ShapeDtypeStruct((NPD, H), jnp.float32),
                  jax.ShapeDtypeStruct((NPD, H), jnp.float32)),
        mesh=mesh,
        scratch_types=[
            pltpu.VMEM_SHARED((NPD, H), jnp.float32),
            pltpu.VMEM((K, C), jnp.int32),
            pltpu.VMEM((K, C), jnp.int32),
            pltpu.VMEM((C, H), jnp.float32),
            pltpu.VMEM((C, H), jnp.float32),
            pltpu.VMEM((C, H), jnp.float32),
            pltpu.VMEM((C, H), jnp.float32),
            pltpu.SemaphoreType.DMA,
            pltpu.SemaphoreType.DMA,
            pltpu.SemaphoreType.DMA,
            pltpu.SemaphoreType.DMA,
            pltpu.SemaphoreType.DMA,
            pltpu.SemaphoreType.DMA,
            pltpu.SemaphoreType.DMA,
        ],
    )
    return fn(a, ep, src2, dst2)


def _zw_kernel(zt_ref, w1_ref, o_ref):
    o_ref[...] = jnp.dot(zt_ref[...], w1_ref[...],
                         preferred_element_type=jnp.float32)


def _zw(z_table_pad, W1):
    return pl.pallas_call(
        _zw_kernel,
        out_shape=jax.ShapeDtypeStruct((ZT, H), jnp.float32),
    )(z_table_pad, W1)


def _ep_kernel(e_ref, w_ref, o_ref):
    o_ref[...] = jnp.dot(e_ref[...], w_ref[...],
                         preferred_element_type=jnp.float32)


def _ep(e_pad, W_edge_l):
    RB = 2048
    return pl.pallas_call(
        _ep_kernel,
        grid=(EP_PAD // RB,),
        in_specs=[
            pl.BlockSpec((RB, 8), lambda r: (r, 0)),
            pl.BlockSpec((8, H), lambda r: (0, 0)),
        ],
        out_specs=pl.BlockSpec((RB, H), lambda r: (r, 0)),
        out_shape=jax.ShapeDtypeStruct((EP_PAD, H), jnp.float32),
    )(e_pad, W_edge_l)


def _oh(ids_ref, width):
    ids = lax.broadcasted_iota(jnp.int32, (R, width), 1)
    return (ids == ids_ref[...]).astype(jnp.float32)


def _dense0_kernel(x_ref, z_ref, b_ref, zw_ref, w2_ref, bi_ref, wcat_ref,
                   a_ref, s_ref, pool_ref):
    i = pl.program_id(0)
    zoh = _oh(z_ref, ZT)
    h0 = (jnp.dot(x_ref[...], w2_ref[...], preferred_element_type=jnp.float32)
          + jnp.dot(zoh, zw_ref[...], preferred_element_type=jnp.float32)
          + bi_ref[...])
    as_ = jnp.dot(h0, wcat_ref[...], preferred_element_type=jnp.float32)
    a_ref[...] = as_[:, :H]
    s_ref[...] = as_[:, H:]
    ohb = _oh(b_ref, GP)
    p = lax.dot_general(ohb, h0, (((0,), (0,)), ((), ())),
                        preferred_element_type=jnp.float32)

    @pl.when(i == 0)
    def _init():
        pool_ref[...] = p

    @pl.when(i > 0)
    def _acc():
        pool_ref[...] += p


def _dense0(x, z2, b2, zw, W2, bi, wcat):
    return pl.pallas_call(
        _dense0_kernel,
        grid=(NB,),
        in_specs=[
            pl.BlockSpec((R, 120), lambda i: (i, 0)),
            pl.BlockSpec((R, 1), lambda i: (i, 0)),
            pl.BlockSpec((R, 1), lambda i: (i, 0)),
            pl.BlockSpec((ZT, H), lambda i: (0, 0)),
            pl.BlockSpec((120, H), lambda i: (0, 0)),
            pl.BlockSpec((1, H), lambda i: (0, 0)),
            pl.BlockSpec((H, 2 * H), lambda i: (0, 0)),
        ],
        out_specs=[
            pl.BlockSpec((R, H), lambda i: (i, 0)),
            pl.BlockSpec((R, H), lambda i: (i, 0)),
            pl.BlockSpec((GP, H), lambda i: (0, 0)),
        ],
        out_shape=[
            jax.ShapeDtypeStruct((NPD, H), jnp.float32),
            jax.ShapeDtypeStruct((N, H), jnp.float32),
            jax.ShapeDtypeStruct((GP, H), jnp.float32),
        ],
    )(x, z2, b2, zw, W2, bi, wcat)


def _ln(hin, g_ref, b_ref):
    mu = jnp.mean(hin, axis=-1, keepdims=True)
    var = jnp.mean((hin - mu) ** 2, axis=-1, keepdims=True)
    return (hin - mu) * lax.rsqrt(var + EPS) * g_ref[...] + b_ref[...]


def _dense_kernel(agg0_ref, agg1_ref, sp_ref, g_ref, lb_ref, vne_ref, b_ref,
                  wcat_ref, a_ref, s_ref, pool_ref):
    i = pl.program_id(0)
    hn = _ln(agg0_ref[...] + agg1_ref[...] + sp_ref[...], g_ref, lb_ref)
    ohb = _oh(b_ref, GP)
    hl = hn + jnp.dot(ohb, vne_ref[...], preferred_element_type=jnp.float32)
    as_ = jnp.dot(hl, wcat_ref[...], preferred_element_type=jnp.float32)
    a_ref[...] = as_[:, :H]
    s_ref[...] = as_[:, H:]
    p = lax.dot_general(ohb, hl, (((0,), (0,)), ((), ())),
                        preferred_element_type=jnp.float32)

    @pl.when(i == 0)
    def _init():
        pool_ref[...] = p

    @pl.when(i > 0)
    def _acc():
        pool_ref[...] += p


def _dense(agg0, agg1, sprev, lg, lb, vne, b2, wcat):
    return pl.pallas_call(
        _dense_kernel,
        grid=(NB,),
        in_specs=[
            pl.BlockSpec((R, H), lambda i: (i, 0)),
            pl.BlockSpec((R, H), lambda i: (i, 0)),
            pl.BlockSpec((R, H), lambda i: (i, 0)),
            pl.BlockSpec((1, H), lambda i: (0, 0)),
            pl.BlockSpec((1, H), lambda i: (0, 0)),
            pl.BlockSpec((GP, H), lambda i: (0, 0)),
            pl.BlockSpec((R, 1), lambda i: (i, 0)),
            pl.BlockSpec((H, 2 * H), lambda i: (0, 0)),
        ],
        out_specs=[
            pl.BlockSpec((R, H), lambda i: (i, 0)),
            pl.BlockSpec((R, H), lambda i: (i, 0)),
            pl.BlockSpec((GP, H), lambda i: (0, 0)),
        ],
        out_shape=[
            jax.ShapeDtypeStruct((NPD, H), jnp.float32),
            jax.ShapeDtypeStruct((N, H), jnp.float32),
            jax.ShapeDtypeStruct((GP, H), jnp.float32),
        ],
    )(agg0, agg1, sprev, lg, lb, vne, b2, wcat)


def _mlp_kernel(pool_ref, vne_ref, w1_ref, b1_ref, w2_ref, b2_ref, o_ref):
    tmp = pool_ref[...] + vne_ref[...]
    t = jax.nn.relu((jnp.dot(tmp, w1_ref[...],
                             preferred_element_type=jnp.float32)
                     + b1_ref[...]) * BNS)
    t = jax.nn.relu((jnp.dot(t, w2_ref[...],
                             preferred_element_type=jnp.float32)
                     + b2_ref[...]) * BNS)
    o_ref[...] = t


def _mlp(pool, vne, w1, b1, w2, b2):
    return pl.pallas_call(
        _mlp_kernel,
        out_shape=jax.ShapeDtypeStruct((GP, H), jnp.float32),
    )(pool, vne, w1, b1, w2, b2)


def _outp_kernel(agg0_ref, agg1_ref, sp_ref, g_ref, lb_ref, wo_ref, bo_ref,
                 o_ref):
    hn = _ln(agg0_ref[...] + agg1_ref[...] + sp_ref[...], g_ref, lb_ref)
    o_ref[...] = jax.nn.relu(
        jnp.dot(hn, wo_ref[...], preferred_element_type=jnp.float32)
        + bo_ref[...])


def _outp(agg0, agg1, sprev, lg, lb, wo, bo):
    return pl.pallas_call(
        _outp_kernel,
        grid=(NB,),
        in_specs=[
            pl.BlockSpec((R, H), lambda i: (i, 0)),
            pl.BlockSpec((R, H), lambda i: (i, 0)),
            pl.BlockSpec((R, H), lambda i: (i, 0)),
            pl.BlockSpec((1, H), lambda i: (0, 0)),
            pl.BlockSpec((1, H), lambda i: (0, 0)),
            pl.BlockSpec((H, H), lambda i: (0, 0)),
            pl.BlockSpec((1, H), lambda i: (0, 0)),
        ],
        out_specs=pl.BlockSpec((R, H), lambda i: (i, 0)),
        out_shape=jax.ShapeDtypeStruct((N, H), jnp.float32),
    )(agg0, agg1, sprev, lg, lb, wo, bo)


def kernel(x, z, edge_index, bond_feature, edge_attr, peripheral_attr, rd, pos,
           batch, z_table, W_init, b_init, W_msg, W_edge, W_self, ln_g, ln_b,
           Wv1, bv1, Wv2, bv2, W_out, b_out):
    npad = EP_PAD - E
    src = edge_index[0].astype(jnp.int32)
    dst = edge_index[1].astype(jnp.int32)
    src_p = jnp.concatenate([src, jnp.zeros((npad,), jnp.int32)])
    dst_p = jnp.concatenate(
        [dst, N + (jnp.arange(npad, dtype=jnp.int32) % (NPD - N))])
    src2 = src_p.reshape(EP_PAD // C, C)
    dst2 = dst_p.reshape(EP_PAD // C, C)
    e = jnp.concatenate([bond_feature, edge_attr], axis=-1)
    e_pad = jnp.pad(e, ((0, npad), (0, 0)))
    z2 = z.astype(jnp.int32)[:, None]
    b2 = batch.astype(jnp.int32)[:, None]
    z_table_pad = jnp.pad(z_table, ((0, ZT - 1000), (0, 0)))

    zw = _zw(z_table_pad, W_init[:8])

    vne = jnp.zeros((GP, H), jnp.float32)
    wcat0 = jnp.concatenate([W_msg[0], W_self[0]], axis=1)
    A, S, pool = _dense0(x, z2, b2, zw, W_init[8:], b_init[None, :], wcat0)

    for l in range(L):
        o0, o1 = _edge_sc(A, _ep(e_pad, W_edge[l]), src2, dst2)
        if l < L - 1:
            vne_next = _mlp(pool, vne, Wv1[l], bv1[l][None, :],
                            Wv2[l], bv2[l][None, :])
            wcat = jnp.concatenate([W_msg[l + 1], W_self[l + 1]], axis=1)
            A, S, pool = _dense(o0, o1, S, ln_g[l][None, :], ln_b[l][None, :],
                                vne_next, b2, wcat)
            vne = vne_next
        else:
            return _outp(o0, o1, S, ln_g[l][None, :], ln_b[l][None, :],
                         W_out, b_out[None, :])


# speedup vs baseline: 1.0341x; 1.0341x over previous
import functools

import jax
import jax.numpy as jnp
import numpy as np
from jax import lax
from jax.experimental import pallas as pl
from jax.experimental.pallas import tpu as pltpu
from jax.experimental.pallas import tpu_sc as plsc

N = 10000
E = 320000
H = 128
L = 4
G = 500
GP = 512
ZT = 1024
R = 400
NB = N // R

NT = 16
NC = 2
C = 64
EP_PAD = 327680
HALF = EP_PAD // 2
EPT0 = 17408
EPT1 = 3072
K = 16
NPD = 10240
RPT = NPD // NT
ZR = 64
BNS = float(1.0 / np.sqrt(1.0 + 1e-5))
EPS = 1e-5


def _edge_body(a, ep, src2, dst2, out0, out1,
               agg_s, sidx, didx, gb0, gb1, pb0, pb1,
               sem_i, sem_g0, sem_g1, sem_e0, sem_e1, sem_s0, sem_s1):
    c = lax.axis_index("c")
    s = lax.axis_index("s")
    gbufs = (gb0, gb1)
    pbufs = (pb0, pb1)
    gsems = (sem_g0, sem_g1)
    esems = (sem_e0, sem_e1)
    ssems = (sem_s0, sem_s1)

    @pl.loop(0, ZR)
    def _zero(i):
        for k in range(8):
            gb0[i, pl.ds(k * 16, 16)] = jnp.zeros((16,), jnp.float32)

    for i in range(RPT // ZR):
        pltpu.sync_copy(gb0.at[pl.ds(0, ZR)],
                        agg_s.at[pl.ds(s * RPT + i * ZR, ZR)])
    plsc.subcore_barrier()

    ebase0 = s * EPT0
    ebase1 = NT * EPT0 + s * EPT1
    ebase = pl.multiple_of(jnp.where(c == 0, ebase0, ebase1), 512)
    nsup = jnp.where(c == 0, EPT0 // (K * C), EPT1 // (K * C))

    def issue_loads(sup, i, b):
        pltpu.async_copy(a.at[sidx.at[i]], gbufs[b], gsems[b])
        pltpu.async_copy(ep.at[pl.ds(ebase + (sup * K + i) * C, C)],
                         pbufs[b], esems[b])

    def wait_loads(b):
        pltpu.make_async_copy(a.at[sidx.at[0]], gbufs[b], gsems[b]).wait()
        pltpu.make_async_copy(ep.at[pl.ds(0, C)], pbufs[b],
                              esems[b]).wait()

    def relu(b):
        gbuf, pbuf = gbufs[b], pbufs[b]

        @pl.loop(0, C)
        def _relu(j):
            for k in range(8):
                sl = pl.ds(k * 16, 16)
                pbuf[j, sl] = jnp.maximum(gbuf[j, sl] + pbuf[j, sl], 0.0)

    def issue_scatter(i, b):
        pltpu.async_copy(pbufs[b], agg_s.at[didx.at[i]], ssems[b], add=True)

    def wait_scatter(b):
        pltpu.make_async_copy(pbufs[b], agg_s.at[didx.at[0]], ssems[b]).wait()

    @pl.loop(0, nsup)
    def _sup(sup):
        irow = pl.multiple_of(ebase // C + sup * K, 8)
        pltpu.sync_copy(src2.at[pl.ds(irow, K)], sidx)
        pltpu.sync_copy(dst2.at[pl.ds(irow, K)], didx)

        issue_loads(sup, 0, 0)
        issue_loads(sup, 1, 1)

        @pl.loop(0, K // 2 - 1)
        def _chunk(ii):
            i = ii * 2
            wait_loads(0)
            relu(0)
            issue_scatter(i, 0)
            wait_loads(1)
            relu(1)
            issue_scatter(i + 1, 1)
            wait_scatter(0)
            issue_loads(sup, i + 2, 0)
            wait_scatter(1)
            issue_loads(sup, i + 3, 1)

        wait_loads(0)
        relu(0)
        issue_scatter(K - 2, 0)
        wait_loads(1)
        relu(1)
        issue_scatter(K - 1, 1)
        wait_scatter(0)
        wait_scatter(1)

    plsc.subcore_barrier()

    @pl.when(c == 0)
    def _o0():
        pltpu.sync_copy(agg_s.at[pl.ds(s * RPT, RPT)],
                        out0.at[pl.ds(s * RPT, RPT)])

    @pl.when(c == 1)
    def _o1():
        pltpu.sync_copy(agg_s.at[pl.ds(s * RPT, RPT)],
                        out1.at[pl.ds(s * RPT, RPT)])


@jax.jit
def _edge_sc(a, ep, src2, dst2):
    mesh = plsc.VectorSubcoreMesh(core_axis_name="c", subcore_axis_name="s")
    fn = pl.kernel(
        _edge_body,
        out_type=(jax.ShapeDtypeStruct((NPD, H), jnp.float32),
                  jax.ShapeDtypeStruct((NPD, H), jnp.float32)),
        mesh=mesh,
        scratch_types=[
            pltpu.VMEM_SHARED((NPD, H), jnp.float32),
            pltpu.VMEM((K, C), jnp.int32),
            pltpu.VMEM((K, C), jnp.int32),
            pltpu.VMEM((C, H), jnp.float32),
            pltpu.VMEM((C, H), jnp.float32),
            pltpu.VMEM((C, H), jnp.float32),
            pltpu.VMEM((C, H), jnp.float32),
            pltpu.SemaphoreType.DMA,
            pltpu.SemaphoreType.DMA,
            pltpu.SemaphoreType.DMA,
            pltpu.SemaphoreType.DMA,
            pltpu.SemaphoreType.DMA,
            pltpu.SemaphoreType.DMA,
            pltpu.SemaphoreType.DMA,
        ],
    )
    return fn(a, ep, src2, dst2)


def _zw_kernel(zt_ref, w1_ref, o_ref):
    o_ref[...] = jnp.dot(zt_ref[...], w1_ref[...],
                         preferred_element_type=jnp.float32)


def _zw(z_table_pad, W1):
    return pl.pallas_call(
        _zw_kernel,
        out_shape=jax.ShapeDtypeStruct((ZT, H), jnp.float32),
    )(z_table_pad, W1)


def _ep_kernel(e_ref, w_ref, o_ref):
    o_ref[...] = jnp.dot(e_ref[...], w_ref[...],
                         preferred_element_type=jnp.float32)


def _ep(e_pad, W_edge_l):
    RB = 2048
    return pl.pallas_call(
        _ep_kernel,
        grid=(EP_PAD // RB,),
        in_specs=[
            pl.BlockSpec((RB, 8), lambda r: (r, 0)),
            pl.BlockSpec((8, H), lambda r: (0, 0)),
        ],
        out_specs=pl.BlockSpec((RB, H), lambda r: (r, 0)),
        out_shape=jax.ShapeDtypeStruct((EP_PAD, H), jnp.float32),
    )(e_pad, W_edge_l)


def _oh(ids_ref, width):
    ids = lax.broadcasted_iota(jnp.int32, (R, width), 1)
    return (ids == ids_ref[...]).astype(jnp.float32)


def _dense0_kernel(x_ref, z_ref, b_ref, zw_ref, w2_ref, bi_ref, wcat_ref,
                   a_ref, s_ref, pool_ref):
    i = pl.program_id(0)
    zoh = _oh(z_ref, ZT)
    h0 = (jnp.dot(x_ref[...], w2_ref[...], preferred_element_type=jnp.float32)
          + jnp.dot(zoh, zw_ref[...], preferred_element_type=jnp.float32)
          + bi_ref[...])
    as_ = jnp.dot(h0, wcat_ref[...], preferred_element_type=jnp.float32)
    a_ref[...] = as_[:, :H]
    s_ref[...] = as_[:, H:]
    ohb = _oh(b_ref, GP)
    p = lax.dot_general(ohb, h0, (((0,), (0,)), ((), ())),
                        preferred_element_type=jnp.float32)

    @pl.when(i == 0)
    def _init():
        pool_ref[...] = p

    @pl.when(i > 0)
    def _acc():
        pool_ref[...] += p


def _dense0(x, z2, b2, zw, W2, bi, wcat):
    return pl.pallas_call(
        _dense0_kernel,
        grid=(NB,),
        in_specs=[
            pl.BlockSpec((R, 120), lambda i: (i, 0)),
            pl.BlockSpec((R, 1), lambda i: (i, 0)),
            pl.BlockSpec((R, 1), lambda i: (i, 0)),
            pl.BlockSpec((ZT, H), lambda i: (0, 0)),
            pl.BlockSpec((120, H), lambda i: (0, 0)),
            pl.BlockSpec((1, H), lambda i: (0, 0)),
            pl.BlockSpec((H, 2 * H), lambda i: (0, 0)),
        ],
        out_specs=[
            pl.BlockSpec((R, H), lambda i: (i, 0)),
            pl.BlockSpec((R, H), lambda i: (i, 0)),
            pl.BlockSpec((GP, H), lambda i: (0, 0)),
        ],
        out_shape=[
            jax.ShapeDtypeStruct((NPD, H), jnp.float32),
            jax.ShapeDtypeStruct((N, H), jnp.float32),
            jax.ShapeDtypeStruct((GP, H), jnp.float32),
        ],
    )(x, z2, b2, zw, W2, bi, wcat)


def _ln(hin, g_ref, b_ref):
    mu = jnp.mean(hin, axis=-1, keepdims=True)
    var = jnp.mean((hin - mu) ** 2, axis=-1, keepdims=True)
    return (hin - mu) * lax.rsqrt(var + EPS) * g_ref[...] + b_ref[...]


def _dense_kernel(agg0_ref, agg1_ref, sp_ref, g_ref, lb_ref, vne_ref, b_ref,
                  wcat_ref, a_ref, s_ref, pool_ref):
    i = pl.program_id(0)
    hn = _ln(agg0_ref[...] + agg1_ref[...] + sp_ref[...], g_ref, lb_ref)
    ohb = _oh(b_ref, GP)
    hl = hn + jnp.dot(ohb, vne_ref[...], preferred_element_type=jnp.float32)
    as_ = jnp.dot(hl, wcat_ref[...], preferred_element_type=jnp.float32)
    a_ref[...] = as_[:, :H]
    s_ref[...] = as_[:, H:]
    p = lax.dot_general(ohb, hl, (((0,), (0,)), ((), ())),
                        preferred_element_type=jnp.float32)

    @pl.when(i == 0)
    def _init():
        pool_ref[...] = p

    @pl.when(i > 0)
    def _acc():
        pool_ref[...] += p


def _dense(agg0, agg1, sprev, lg, lb, vne, b2, wcat):
    return pl.pallas_call(
        _dense_kernel,
        grid=(NB,),
        in_specs=[
            pl.BlockSpec((R, H), lambda i: (i, 0)),
            pl.BlockSpec((R, H), lambda i: (i, 0)),
            pl.BlockSpec((R, H), lambda i: (i, 0)),
            pl.BlockSpec((1, H), lambda i: (0, 0)),
            pl.BlockSpec((1, H), lambda i: (0, 0)),
            pl.BlockSpec((GP, H), lambda i: (0, 0)),
            pl.BlockSpec((R, 1), lambda i: (i, 0)),
            pl.BlockSpec((H, 2 * H), lambda i: (0, 0)),
        ],
        out_specs=[
            pl.BlockSpec((R, H), lambda i: (i, 0)),
            pl.BlockSpec((R, H), lambda i: (i, 0)),
            pl.BlockSpec((GP, H), lambda i: (0, 0)),
        ],
        out_shape=[
            jax.ShapeDtypeStruct((NPD, H), jnp.float32),
            jax.ShapeDtypeStruct((N, H), jnp.float32),
            jax.ShapeDtypeStruct((GP, H), jnp.float32),
        ],
    )(agg0, agg1, sprev, lg, lb, vne, b2, wcat)


def _mlp_kernel(pool_ref, vne_ref, w1_ref, b1_ref, w2_ref, b2_ref, o_ref):
    tmp = pool_ref[...] + vne_ref[...]
    t = jax.nn.relu((jnp.dot(tmp, w1_ref[...],
                             preferred_element_type=jnp.float32)
                     + b1_ref[...]) * BNS)
    t = jax.nn.relu((jnp.dot(t, w2_ref[...],
                             preferred_element_type=jnp.float32)
                     + b2_ref[...]) * BNS)
    o_ref[...] = t


def _mlp(pool, vne, w1, b1, w2, b2):
    return pl.pallas_call(
        _mlp_kernel,
        out_shape=jax.ShapeDtypeStruct((GP, H), jnp.float32),
    )(pool, vne, w1, b1, w2, b2)


def _outp_kernel(agg0_ref, agg1_ref, sp_ref, g_ref, lb_ref, wo_ref, bo_ref,
                 o_ref):
    hn = _ln(agg0_ref[...] + agg1_ref[...] + sp_ref[...], g_ref, lb_ref)
    o_ref[...] = jax.nn.relu(
        jnp.dot(hn, wo_ref[...], preferred_element_type=jnp.float32)
        + bo_ref[...])


def _outp(agg0, agg1, sprev, lg, lb, wo, bo):
    return pl.pallas_call(
        _outp_kernel,
        grid=(NB,),
        in_specs=[
            pl.BlockSpec((R, H), lambda i: (i, 0)),
            pl.BlockSpec((R, H), lambda i: (i, 0)),
            pl.BlockSpec((R, H), lambda i: (i, 0)),
            pl.BlockSpec((1, H), lambda i: (0, 0)),
            pl.BlockSpec((1, H), lambda i: (0, 0)),
            pl.BlockSpec((H, H), lambda i: (0, 0)),
            pl.BlockSpec((1, H), lambda i: (0, 0)),
        ],
        out_specs=pl.BlockSpec((R, H), lambda i: (i, 0)),
        out_shape=jax.ShapeDtypeStruct((N, H), jnp.float32),
    )(agg0, agg1, sprev, lg, lb, wo, bo)


def kernel(x, z, edge_index, bond_feature, edge_attr, peripheral_attr, rd, pos,
           batch, z_table, W_init, b_init, W_msg, W_edge, W_self, ln_g, ln_b,
           Wv1, bv1, Wv2, bv2, W_out, b_out):
    npad = EP_PAD - E
    src = edge_index[0].astype(jnp.int32)
    dst = edge_index[1].astype(jnp.int32)
    src_p = jnp.concatenate([src, jnp.zeros((npad,), jnp.int32)])
    dst_p = jnp.concatenate(
        [dst, N + (jnp.arange(npad, dtype=jnp.int32) % (NPD - N))])
    src2 = src_p.reshape(EP_PAD // C, C)
    dst2 = dst_p.reshape(EP_PAD // C, C)
    e = jnp.concatenate([bond_feature, edge_attr], axis=-1)
    e_pad = jnp.pad(e, ((0, npad), (0, 0)))
    z2 = z.astype(jnp.int32)[:, None]
    b2 = batch.astype(jnp.int32)[:, None]
    z_table_pad = jnp.pad(z_table, ((0, ZT - 1000), (0, 0)))

    zw = _zw(z_table_pad, W_init[:8])

    vne = jnp.zeros((GP, H), jnp.float32)
    wcat0 = jnp.concatenate([W_msg[0], W_self[0]], axis=1)
    A, S, pool = _dense0(x, z2, b2, zw, W_init[8:], b_init[None, :], wcat0)

    for l in range(L):
        o0, o1 = _edge_sc(A, _ep(e_pad, W_edge[l]), src2, dst2)
        if l < L - 1:
            vne_next = _mlp(pool, vne, Wv1[l], bv1[l][None, :],
                            Wv2[l], bv2[l][None, :])
            wcat = jnp.concatenate([W_msg[l + 1], W_self[l + 1]], axis=1)
            A, S, pool = _dense(o0, o1, S, ln_g[l][None, :], ln_b[l][None, :],
                                vne_next, b2, wcat)
            vne = vne_next
        else:
            return _outp(o0, o1, S, ln_g[l][None, :], ln_b[l][None, :],
                         W_out, b_out[None, :])
